# Initial kernel scaffold; baseline (speedup 1.0000x reference)
import jax
import jax.numpy as jnp
from jax.experimental import pallas as pl


def kernel(q, k, v, Wq, Wk, Wv, Wo):
    raise NotImplementedError("write your pallas kernel here")


# speedup vs baseline: 55.4687x; 55.4687x over previous
import jax
import jax.numpy as jnp
from jax.experimental import pallas as pl
from jax.experimental.pallas import tpu as pltpu

HEADS = 12
DQK = 64
DV = 64
STRIDE = 4
SPAN = 128
TILE_Q = 128
BLK = TILE_Q // STRIDE
WIN = 2 * BLK
KPAD = BLK


def _body(q_ref, k_ref, v_ref, wqt_ref, wkt_ref, wvt_ref, wot_ref,
          out_ref, kp_ref, vp_ref):
    t = pl.program_id(0)

    @pl.when(t == 0)
    def _project_kv():
        kp_ref[0:KPAD, :] = jnp.zeros((KPAD, kp_ref.shape[1]), jnp.float32)
        vp_ref[0:KPAD, :] = jnp.zeros((KPAD, vp_ref.shape[1]), jnp.float32)
        kp_ref[KPAD:, :] = jnp.dot(k_ref[...], wkt_ref[...],
                                   preferred_element_type=jnp.float32)
        vp_ref[KPAD:, :] = jnp.dot(v_ref[...], wvt_ref[...],
                                   preferred_element_type=jnp.float32)

    qp = jnp.dot(q_ref[...], wqt_ref[...], preferred_element_type=jnp.float32)
    kwin = kp_ref[pl.ds(t * BLK, WIN), :]
    vwin = vp_ref[pl.ds(t * BLK, WIN), :]

    i = jax.lax.broadcasted_iota(jnp.int32, (TILE_Q, WIN), 0)
    m = jax.lax.broadcasted_iota(jnp.int32, (TILE_Q, WIN), 1)
    g = i >> 2
    valid = (m >= g + 1) & (m <= g + BLK) & (m + t * BLK >= BLK)

    scale = 1.0 / (DQK ** 0.5)
    outs = []
    for h in range(HEADS):
        qh = qp[:, h * DQK:(h + 1) * DQK]
        kh = kwin[:, h * DQK:(h + 1) * DQK]
        vh = vwin[:, h * DV:(h + 1) * DV]
        s = jax.lax.dot_general(qh, kh, (((1,), (1,)), ((), ())),
                                preferred_element_type=jnp.float32) * scale
        s = jnp.where(valid, s, -1e30)
        e = jnp.exp(s - jnp.max(s, axis=1, keepdims=True))
        p = e / jnp.sum(e, axis=1, keepdims=True)
        outs.append(jnp.dot(p, vh, preferred_element_type=jnp.float32))
    attn = jnp.concatenate(outs, axis=1)
    out_ref[...] = jnp.dot(attn, wot_ref[...], preferred_element_type=jnp.float32)


def kernel(q, k, v, Wq, Wk, Wv, Wo):
    batch, len_q, dim_q = q.shape
    dim_k = k.shape[2]
    dim_vin = v.shape[2]
    dim_out = Wo.shape[0]
    kmax = ((len_q - 1) // STRIDE) + 1
    kmax = ((kmax + BLK - 1) // BLK) * BLK

    q2 = q.reshape(batch * len_q, dim_q)
    k2 = k[0, :kmax, :]
    v2 = v[0, :kmax, :]

    grid = (len_q // TILE_Q,)
    out = pl.pallas_call(
        _body,
        grid=grid,
        in_specs=[
            pl.BlockSpec((TILE_Q, dim_q), lambda t: (t, 0)),
            pl.BlockSpec((kmax, dim_k), lambda t: (0, 0)),
            pl.BlockSpec((kmax, dim_vin), lambda t: (0, 0)),
            pl.BlockSpec((dim_q, HEADS * DQK), lambda t: (0, 0)),
            pl.BlockSpec((dim_k, HEADS * DQK), lambda t: (0, 0)),
            pl.BlockSpec((dim_vin, HEADS * DV), lambda t: (0, 0)),
            pl.BlockSpec((HEADS * DV, dim_out), lambda t: (0, 0)),
        ],
        out_specs=pl.BlockSpec((TILE_Q, dim_out), lambda t: (t, 0)),
        out_shape=jax.ShapeDtypeStruct((len_q, dim_out), jnp.float32),
        scratch_shapes=[
            pltpu.VMEM((KPAD + kmax, HEADS * DQK), jnp.float32),
            pltpu.VMEM((KPAD + kmax, HEADS * DV), jnp.float32),
        ],
    )(q2, k2, v2, Wq.T, Wk.T, Wv.T, Wo.T)
    return out.reshape(batch, len_q, dim_out)
